# Initial kernel scaffold; baseline (speedup 1.0000x reference)
import jax
import jax.numpy as jnp
from jax.experimental import pallas as pl


def kernel(context, time_table, week_table, season_table):
    raise NotImplementedError("write your pallas kernel here")


# speedup vs baseline: 2.3433x; 2.3433x over previous
import functools

import jax
import jax.numpy as jnp
from jax import lax
from jax.experimental import pallas as pl
from jax.experimental.pallas import tpu as pltpu
from jax.experimental.pallas import tpu_sc as plsc

_B, _S = 16384, 200
_P = _B * _S
_DT, _DW, _DS = 8, 4, 6
_D = _DT + _DW + _DS
_NT, _NW_TBL, _NS_TBL = 6, 2, 4
_NFUSED = _NT * _NW_TBL * _NS_TBL
_L = 16
_NWORKERS = 32
_PT = _P // _NWORKERS
_CP = 2048
_NCHUNK = _PT // _CP
_GROUPS = _CP // 8


def _tec_body(ctx_hbm, fused_hbm, out_hbm, ctx_v, c18_v, out_v, fused_v):
    wid = lax.axis_index("s") * 2 + lax.axis_index("c")
    lane = lax.iota(jnp.int32, _L)

    pltpu.sync_copy(fused_hbm, fused_v)

    relp = []
    kpat = []
    for j in range(9):
        v = j * _L + lane
        p = v // _D
        relp.append(p)
        kpat.append(v - p * _D)

    s0 = lane * 3

    def chunk_body(it, carry):
        base = wid * _PT + it * _CP
        pltpu.sync_copy(ctx_hbm.at[pl.ds(base * 3, _CP * 3)], ctx_v)

        def cbuf_body(i, c):
            b = i * (_L * 3)
            i0 = plsc.load_gather(ctx_v, [b + s0])
            i1 = plsc.load_gather(ctx_v, [b + s0 + 1])
            i2 = plsc.load_gather(ctx_v, [b + s0 + 2])
            i0 = jnp.clip(i0, 0, _NT - 1)
            i1 = jnp.clip(i1, 0, _NW_TBL - 1)
            i2 = jnp.clip(i2, 0, _NS_TBL - 1)
            c18_v[pl.ds(i * _L, _L)] = i0 * _D + i1 * (_NT * _D) + i2 * (_NT * _NW_TBL * _D)
            return c
        lax.fori_loop(0, _CP // _L, cbuf_body, 0, unroll=2)

        def group_body(g, c):
            pos0 = g * 8
            obase = g * 144
            for j in range(9):
                cg = plsc.load_gather(c18_v, [pos0 + relp[j]])
                out_v[pl.ds(obase + j * _L, _L)] = plsc.load_gather(fused_v, [cg + kpat[j]])
            return c
        lax.fori_loop(0, _GROUPS, group_body, 0)

        pltpu.sync_copy(out_v, out_hbm.at[pl.ds(base * _D, _CP * _D)])
        return carry

    lax.fori_loop(0, _NCHUNK, chunk_body, 0)


@functools.cache
def _sc_lookup():
    return functools.partial(
        pl.kernel,
        mesh=plsc.VectorSubcoreMesh(core_axis_name="c", subcore_axis_name="s"),
        compiler_params=pltpu.CompilerParams(needs_layout_passes=False),
        out_type=jax.ShapeDtypeStruct((_P * _D,), jnp.float32),
        scratch_types=[
            pltpu.VMEM((_CP * 3,), jnp.int32),
            pltpu.VMEM((_CP,), jnp.int32),
            pltpu.VMEM((_CP * _D,), jnp.float32),
            pltpu.VMEM((_NFUSED * _D,), jnp.float32),
        ],
    )(_tec_body)


def kernel(context, time_table, week_table, season_table):
    ctx = context.reshape(-1).astype(jnp.int32)
    t = jnp.broadcast_to(time_table[None, None], (_NS_TBL, _NW_TBL, _NT, _DT))
    w = jnp.broadcast_to(week_table[None, :, None], (_NS_TBL, _NW_TBL, _NT, _DW))
    s = jnp.broadcast_to(season_table[:, None, None], (_NS_TBL, _NW_TBL, _NT, _DS))
    fused = jnp.concatenate((t, w, s), axis=-1).reshape(-1)
    out = _sc_lookup()(ctx, fused)
    return out.reshape(_B, _S, _D)


# speedup vs baseline: 53.9112x; 23.0065x over previous
import functools

import jax
import jax.numpy as jnp
from jax import lax
from jax.experimental import pallas as pl
from jax.experimental.pallas import tpu as pltpu
from jax.experimental.pallas import tpu_sc as plsc

_B, _S = 16384, 200
_DT, _DW, _DS = 8, 4, 6
_D = _DT + _DW + _DS
_NT, _NW_TBL, _NS_TBL = 6, 2, 4
_NFUSED = _NT * _NW_TBL * _NS_TBL
_L = 16
_NWORKERS = 32
_W = 256
_ST = _S // 8
_BC = _B // _W
_NCHUNK = _ST * _BC
_CPW = _NCHUNK // _NWORKERS
_GROUPS = 8 * _W // _L


def _tec_body(ctx_hbm, fused_hbm, out_hbm, ctx_v, o_v, fused_v):
    wid = lax.axis_index("s") * 2 + lax.axis_index("c")

    pltpu.sync_copy(fused_hbm, fused_v)

    def chunk_body(it, carry):
        q = wid * _CPW + it
        s0 = (q // _BC) * 8
        b0 = (q % _BC) * _W
        for k in range(3):
            pltpu.sync_copy(ctx_hbm.at[k, pl.ds(s0, 8), pl.ds(b0, _W)],
                            ctx_v.at[k])

        def group_body(g, c):
            r = g >> 4
            cc = (g & 15) * _L
            i0 = ctx_v[0, r, pl.ds(cc, _L)]
            i1 = ctx_v[1, r, pl.ds(cc, _L)]
            i2 = ctx_v[2, r, pl.ds(cc, _L)]
            i0 = jnp.clip(i0, 0, _NT - 1)
            i1 = jnp.clip(i1, 0, _NW_TBL - 1)
            i2 = jnp.clip(i2, 0, _NS_TBL - 1)
            c18 = i0 * _D + i1 * (_NT * _D) + i2 * (_NT * _NW_TBL * _D)
            for f in range(_D):
                o_v[f, r, pl.ds(cc, _L)] = plsc.load_gather(fused_v, [c18 + f])
            return c
        lax.fori_loop(0, _GROUPS, group_body, 0)

        for f in range(_D):
            pltpu.sync_copy(o_v.at[f],
                            out_hbm.at[f, pl.ds(s0, 8), pl.ds(b0, _W)])
        return carry

    lax.fori_loop(0, _CPW, chunk_body, 0)


@functools.cache
def _sc_lookup():
    return functools.partial(
        pl.kernel,
        mesh=plsc.VectorSubcoreMesh(core_axis_name="c", subcore_axis_name="s"),
        compiler_params=pltpu.CompilerParams(needs_layout_passes=False),
        out_type=jax.ShapeDtypeStruct((_D, _S, _B), jnp.float32),
        scratch_types=[
            pltpu.VMEM((3, 8, _W), jnp.int32),
            pltpu.VMEM((_D, 8, _W), jnp.float32),
            pltpu.VMEM((_NFUSED * _D,), jnp.float32),
        ],
    )(_tec_body)


def kernel(context, time_table, week_table, season_table):
    ctx_t = jnp.transpose(context.astype(jnp.int32), (2, 1, 0))
    t = jnp.broadcast_to(time_table[None, None], (_NS_TBL, _NW_TBL, _NT, _DT))
    w = jnp.broadcast_to(week_table[None, :, None], (_NS_TBL, _NW_TBL, _NT, _DW))
    s = jnp.broadcast_to(season_table[:, None, None], (_NS_TBL, _NW_TBL, _NT, _DS))
    fused = jnp.concatenate((t, w, s), axis=-1).reshape(-1)
    out_t = _sc_lookup()(ctx_t, fused)
    return jnp.transpose(out_t, (2, 1, 0))


# speedup vs baseline: 242.3407x; 4.4952x over previous
import functools

import jax
import jax.numpy as jnp
from jax import lax
from jax.experimental import pallas as pl
from jax.experimental.pallas import tpu as pltpu
from jax.experimental.pallas import tpu_sc as plsc

_B, _S = 16384, 200
_DT, _DW, _DS = 8, 4, 6
_D = _DT + _DW + _DS
_NT, _NW_TBL, _NS_TBL = 6, 2, 4
_NFUSED = _NT * _NW_TBL * _NS_TBL
_L = 16
_NWORKERS = 32
_W = 256
_ST = _S // 8
_BC = _B // _W
_NCHUNK = _ST * _BC
_CPW = _NCHUNK // _NWORKERS
_GROUPS = 8 * _W // _L


def _tec_body(ctx_hbm, fused_hbm, out_hbm, ctx_v, o_v, fused_v,
              sem_in0, sem_in1, sem_out0, sem_out1):
    wid = lax.axis_index("s") * 2 + lax.axis_index("c")
    sem_in = (sem_in0, sem_in1)
    sem_out = (sem_out0, sem_out1)

    pltpu.sync_copy(fused_hbm, fused_v)

    def in_copies(q, b):
        s0 = (q // _BC) * 8
        b0 = (q % _BC) * _W
        return [pltpu.make_async_copy(
                    ctx_hbm.at[k, pl.ds(s0, 8), pl.ds(b0, _W)],
                    ctx_v.at[b, k], sem_in[b])
                for k in range(3)]

    def out_copies(q, b):
        s0 = (q // _BC) * 8
        b0 = (q % _BC) * _W
        return [pltpu.make_async_copy(
                    o_v.at[b, f],
                    out_hbm.at[f, pl.ds(s0, 8), pl.ds(b0, _W)], sem_out[b])
                for f in range(_D)]

    def compute(b):
        def group_body(g, c):
            r = g >> 4
            cc = (g & 15) * _L
            i0 = ctx_v[b, 0, r, pl.ds(cc, _L)]
            i1 = ctx_v[b, 1, r, pl.ds(cc, _L)]
            i2 = ctx_v[b, 2, r, pl.ds(cc, _L)]
            i0 = jnp.clip(i0, 0, _NT - 1)
            i1 = jnp.clip(i1, 0, _NW_TBL - 1)
            i2 = jnp.clip(i2, 0, _NS_TBL - 1)
            c18 = i0 * _D + i1 * (_NT * _D) + i2 * (_NT * _NW_TBL * _D)
            vals = [plsc.load_gather(fused_v, [c18 + f]) for f in range(_D)]
            for f in range(_D):
                o_v[b, f, r, pl.ds(cc, _L)] = vals[f]
            return c
        lax.fori_loop(0, _GROUPS, group_body, 0)

    q0 = wid * _CPW
    for d in in_copies(q0, 0):
        d.start()

    def outer_body(j, carry):
        for s in (0, 1):
            q = q0 + j * 2 + s
            if s == 0:
                for d in in_copies(q + 1, 1):
                    d.start()
            else:
                @pl.when(j < (_CPW // 2) - 1)
                def _():
                    for d in in_copies(q + 1, 0):
                        d.start()
            @pl.when(j >= 1)
            def _():
                for d in out_copies(q - 2, s):
                    d.wait()
            for d in in_copies(q, s):
                d.wait()
            compute(s)
            for d in out_copies(q, s):
                d.start()
        return carry

    lax.fori_loop(0, _CPW // 2, outer_body, 0)
    for d in out_copies(q0 + _CPW - 2, 0):
        d.wait()
    for d in out_copies(q0 + _CPW - 1, 1):
        d.wait()


@functools.cache
def _sc_lookup():
    return functools.partial(
        pl.kernel,
        mesh=plsc.VectorSubcoreMesh(core_axis_name="c", subcore_axis_name="s"),
        compiler_params=pltpu.CompilerParams(needs_layout_passes=False),
        out_type=jax.ShapeDtypeStruct((_D, _S, _B), jnp.float32),
        scratch_types=[
            pltpu.VMEM((2, 3, 8, _W), jnp.int32),
            pltpu.VMEM((2, _D, 8, _W), jnp.float32),
            pltpu.VMEM((_NFUSED * _D,), jnp.float32),
            pltpu.SemaphoreType.DMA,
            pltpu.SemaphoreType.DMA,
            pltpu.SemaphoreType.DMA,
            pltpu.SemaphoreType.DMA,
        ],
    )(_tec_body)


def kernel(context, time_table, week_table, season_table):
    ctx_t = jnp.transpose(context.astype(jnp.int32), (2, 1, 0))
    t = jnp.broadcast_to(time_table[None, None], (_NS_TBL, _NW_TBL, _NT, _DT))
    w = jnp.broadcast_to(week_table[None, :, None], (_NS_TBL, _NW_TBL, _NT, _DW))
    s = jnp.broadcast_to(season_table[:, None, None], (_NS_TBL, _NW_TBL, _NT, _DS))
    fused = jnp.concatenate((t, w, s), axis=-1).reshape(-1)
    out_t = _sc_lookup()(ctx_t, fused)
    return jnp.transpose(out_t, (2, 1, 0))
